# baseline (speedup 1.0000x reference)
import jax
import jax.numpy as jnp
from jax import lax
from jax.experimental import pallas as pl
from jax.experimental.pallas import tpu as pltpu
from jax.experimental.pallas import tpu_sc as plsc

BATCH = 16384
HIST = 50
D = 64
NC = 2
NS = 16
NW = NC * NS
E = BATCH * HIST
EPW = E // NW
K = 128
NCH = EPW // K
NBUF = 4


def _gather_body(idx_hbm, table_hbm, out_hbm, idx_v, rows_v, sem_g, sem_o):
  wid = lax.axis_index("s") * NC + lax.axis_index("c")
  pltpu.sync_copy(idx_hbm.at[pl.ds(wid * NCH, NCH)], idx_v)

  def gather_piece(c, b, j):
    idx16 = idx_v[c, pl.ds(j * 16, 16)]
    return pltpu.make_async_copy(
        table_hbm.at[idx16], rows_v.at[b, pl.ds(j * 16, 16)], sem_g.at[b])

  def gathers_start(c, b):
    for j in range(K // 16):
      gather_piece(c, b, j).start()

  def gathers_wait(c, b):
    for j in range(K // 16):
      gather_piece(c, b, j).wait()

  def store(c, b):
    return pltpu.make_async_copy(
        rows_v.at[b], out_hbm.at[pl.ds(wid * EPW + c * K, K)], sem_o.at[b])

  for b in range(NBUF):
    gathers_start(b, b)

  def outer(i, carry):
    c0 = i * NBUF
    for b in range(NBUF):
      gathers_wait(c0 + b, b)
      store(c0 + b, b).start()
    for b in range(NBUF):
      store(c0 + b, b).wait()

      @pl.when(i + 1 < NCH // NBUF)
      def _():
        gathers_start(c0 + b + NBUF, b)

    return carry

  lax.fori_loop(0, NCH // NBUF, outer, 0)


@jax.jit
def _embed(idx, table):
  mesh = plsc.VectorSubcoreMesh(core_axis_name="c", subcore_axis_name="s")
  f = pl.kernel(
      _gather_body,
      out_type=jax.ShapeDtypeStruct((E, D), jnp.float32),
      mesh=mesh,
      scratch_types=[
          pltpu.VMEM((NCH, K), jnp.int32),
          pltpu.VMEM((NBUF, K, D), jnp.float32),
          pltpu.SemaphoreType.DMA((NBUF,)),
          pltpu.SemaphoreType.DMA((NBUF,)),
      ],
      compiler_params=pltpu.CompilerParams(use_tc_tiling_on_sc=False),
  )
  return f(idx.reshape(E // K, K), table).reshape(BATCH, HIST, D)


def kernel(input, C):
  return _embed(input, C)


# speedup vs baseline: 1.0064x; 1.0064x over previous
import jax
import jax.numpy as jnp
from jax import lax
from jax.experimental import pallas as pl
from jax.experimental.pallas import tpu as pltpu
from jax.experimental.pallas import tpu_sc as plsc

BATCH = 16384
HIST = 50
D = 64
NC = 2
NS = 16
NW = NC * NS
E = BATCH * HIST
EPW = E // NW
K = 128
NCH = EPW // K
NBUF = 8


def _gather_body(idx_hbm, table_hbm, out_hbm, idx_v, rows_v, sem_g, sem_o):
  wid = lax.axis_index("s") * NC + lax.axis_index("c")
  pltpu.sync_copy(idx_hbm.at[pl.ds(wid * NCH, NCH)], idx_v)

  def gather_piece(c, b, j):
    idx16 = idx_v[c, pl.ds(j * 16, 16)]
    return pltpu.make_async_copy(
        table_hbm.at[idx16], rows_v.at[b, pl.ds(j * 16, 16)], sem_g.at[b])

  def gathers_start(c, b):
    for j in range(K // 16):
      gather_piece(c, b, j).start()

  def gathers_wait(c, b):
    for j in range(K // 16):
      gather_piece(c, b, j).wait()

  def store(c, b):
    return pltpu.make_async_copy(
        rows_v.at[b], out_hbm.at[pl.ds(wid * EPW + c * K, K)], sem_o.at[b])

  for b in range(NBUF):
    gathers_start(b, b)

  def outer(i, carry):
    c0 = i * NBUF
    for b in range(NBUF):
      gathers_wait(c0 + b, b)
      store(c0 + b, b).start()
    for b in range(NBUF):
      store(c0 + b, b).wait()

      @pl.when(i + 1 < NCH // NBUF)
      def _():
        gathers_start(c0 + b + NBUF, b)

    return carry

  lax.fori_loop(0, NCH // NBUF, outer, 0)


@jax.jit
def _embed(idx, table):
  mesh = plsc.VectorSubcoreMesh(core_axis_name="c", subcore_axis_name="s")
  f = pl.kernel(
      _gather_body,
      out_type=jax.ShapeDtypeStruct((E, D), jnp.float32),
      mesh=mesh,
      scratch_types=[
          pltpu.VMEM((NCH, K), jnp.int32),
          pltpu.VMEM((NBUF, K, D), jnp.float32),
          pltpu.SemaphoreType.DMA((NBUF,)),
          pltpu.SemaphoreType.DMA((NBUF,)),
      ],
      compiler_params=pltpu.CompilerParams(use_tc_tiling_on_sc=False),
  )
  return f(idx.reshape(E // K, K), table).reshape(BATCH, HIST, D)


def kernel(input, C):
  return _embed(input, C)
